# baseline (speedup 1.0000x reference)
import jax
import jax.numpy as jnp
from jax import lax
from jax.experimental import pallas as pl
from jax.experimental.pallas import tpu as pltpu
from jax.experimental.pallas import tpu_sc as plsc

N_ATOMS = 100000
N_SEG = 4096
N_TAB = 119

NUM_TILES = 16
CHUNKS = 49
PER_TILE = CHUNKS * 128
LAST = NUM_TILES - 1
LAST_BASE = LAST * PER_TILE
LAST_N = N_ATOMS - LAST_BASE
LAST_FULL = LAST_N // 128
LAST_REM = LAST_N - LAST_FULL * 128
LAST_CHUNKS = LAST_FULL + 1
ACC = 4352
ACC_PER_TILE = ACC // NUM_TILES
OUT_PER_TILE = N_SEG // NUM_TILES
WINDOW = 32


def _body(energy_hbm, natoms_hbm, z_hbm, seg_hbm, sc_hbm, sh_hbm, table_hbm,
          out_hbm,
          table_v, z_v, seg_v, vals_v, zero_v, acc_sh,
          e_v, na_v, acc_v, res_v, s_v, sh_v, sem, sem2):
    t = lax.axis_index("s")
    base = t * PER_TILE

    @pl.when(t < LAST)
    def _():
        pltpu.async_copy(z_hbm.at[pl.ds(base, PER_TILE)], z_v, sem2)

        def stage(j, c):
            pltpu.async_copy(seg_hbm.at[pl.ds(base + j * 128, 128)],
                             seg_v.at[j], sem2)
            return c

        lax.fori_loop(0, CHUNKS, stage, 0)

    @pl.when(t == LAST)
    def _():
        pltpu.async_copy(z_hbm.at[pl.ds(LAST_BASE, LAST_N)],
                         z_v.at[pl.ds(0, LAST_N)], sem2)

        def stage(j, c):
            pltpu.async_copy(seg_hbm.at[pl.ds(LAST_BASE + j * 128, 128)],
                             seg_v.at[j], sem2)
            return c

        lax.fori_loop(0, LAST_FULL, stage, 0)
        pltpu.async_copy(seg_hbm.at[pl.ds(LAST_BASE + LAST_FULL * 128,
                                          LAST_REM)],
                         seg_v.at[LAST_FULL, pl.ds(0, LAST_REM)], sem2)

    pltpu.sync_copy(table_hbm, table_v)
    obase = t * OUT_PER_TILE
    pltpu.sync_copy(energy_hbm.at[pl.ds(obase, OUT_PER_TILE)], e_v)
    pltpu.sync_copy(natoms_hbm.at[pl.ds(obase, OUT_PER_TILE)], na_v)
    pltpu.sync_copy(sc_hbm, s_v)
    pltpu.sync_copy(sh_hbm, sh_v)

    for i in range(ACC_PER_TILE // 16):
        zero_v[pl.ds(i * 16, 16)] = jnp.zeros((16,), jnp.float32)
    pltpu.sync_copy(zero_v, acc_sh.at[pl.ds(t * ACC_PER_TILE, ACC_PER_TILE)])

    @pl.when(t < LAST)
    def _():
        pltpu.make_async_copy(z_hbm.at[pl.ds(0, PER_TILE)], z_v, sem2).wait()
        pltpu.make_async_copy(z_hbm.at[pl.ds(0, PER_TILE)], z_v, sem2).wait()

    @pl.when(t == LAST)
    def _():
        pltpu.make_async_copy(z_hbm.at[pl.ds(0, LAST_N)],
                              z_v.at[pl.ds(0, LAST_N)], sem2).wait()
        pltpu.make_async_copy(z_hbm.at[pl.ds(0, LAST_N)],
                              z_v.at[pl.ds(0, LAST_N)], sem2).wait()
        sink16 = jnp.full((16,), N_SEG, jnp.int32)
        zi16 = jnp.zeros((16,), jnp.int32)
        for k in range(LAST_REM // 16, 8):
            z_v[pl.ds(LAST_FULL * 128 + k * 16, 16)] = zi16
            seg_v[LAST_FULL, pl.ds(k * 16, 16)] = sink16

    plsc.subcore_barrier()

    trip = jnp.where(t == LAST, LAST_CHUNKS, CHUNKS)

    def chunk(j, c):
        for k in range(8):
            o = j * 128 + k * 16
            idx = z_v[pl.ds(o, 16)]
            vals_v[pl.ds(o, 16)] = plsc.load_gather(table_v, [idx])
        pltpu.async_copy(vals_v.at[pl.ds(j * 128, 128)],
                         acc_sh.at[seg_v.at[j]], sem, add=True)

        @pl.when(j >= WINDOW)
        def _():
            pltpu.make_async_copy(energy_hbm.at[pl.ds(0, 128)],
                                  vals_v.at[pl.ds(0, 128)], sem).wait()

        return c

    lax.fori_loop(0, trip, chunk, 0)
    for _ in range(WINDOW):
        pltpu.make_async_copy(energy_hbm.at[pl.ds(0, 128)],
                              vals_v.at[pl.ds(0, 128)], sem).wait()

    plsc.subcore_barrier()

    pltpu.sync_copy(acc_sh.at[pl.ds(obase, OUT_PER_TILE)], acc_v)
    zero_i16 = jnp.zeros((16,), jnp.int32)
    s = plsc.load_gather(s_v, [zero_i16])
    sh = plsc.load_gather(sh_v, [zero_i16])
    for i in range(OUT_PER_TILE // 16):
        d = pl.ds(i * 16, 16)
        res_v[d] = e_v[d] * s + na_v[d].astype(jnp.float32) * sh + acc_v[d]
    pltpu.sync_copy(res_v, out_hbm.at[pl.ds(obase, OUT_PER_TILE)])


@jax.jit
def _run(energy, n_atoms, z, seg, scale_by, shift_by, table):
    mesh = plsc.VectorSubcoreMesh(core_axis_name="c", subcore_axis_name="s",
                                  num_cores=1)
    return pl.kernel(
        _body,
        out_type=jax.ShapeDtypeStruct((N_SEG,), jnp.float32),
        mesh=mesh,
        compiler_params=pltpu.CompilerParams(needs_layout_passes=False),
        scratch_types=[
            pltpu.VMEM((N_TAB,), jnp.float32),
            pltpu.VMEM((PER_TILE,), jnp.int32),
            pltpu.VMEM((CHUNKS, 128), jnp.int32),
            pltpu.VMEM((PER_TILE,), jnp.float32),
            pltpu.VMEM((ACC_PER_TILE,), jnp.float32),
            pltpu.VMEM_SHARED((ACC,), jnp.float32),
            pltpu.VMEM((OUT_PER_TILE,), jnp.float32),
            pltpu.VMEM((OUT_PER_TILE,), jnp.int32),
            pltpu.VMEM((OUT_PER_TILE,), jnp.float32),
            pltpu.VMEM((OUT_PER_TILE,), jnp.float32),
            pltpu.VMEM((1,), jnp.float32),
            pltpu.VMEM((1,), jnp.float32),
            pltpu.SemaphoreType.DMA,
            pltpu.SemaphoreType.DMA,
        ],
    )(energy, n_atoms, z, seg, scale_by, shift_by, table)


def kernel(energy, n_atoms, Z, image_idx, scale_by, shift_by, atomic_energies):
    return _run(energy, n_atoms, Z.astype(jnp.int32),
                image_idx.astype(jnp.int32), scale_by, shift_by,
                atomic_energies)


# speedup vs baseline: 1.0396x; 1.0396x over previous
import jax
import jax.numpy as jnp
from jax import lax
from jax.experimental import pallas as pl
from jax.experimental.pallas import tpu as pltpu
from jax.experimental.pallas import tpu_sc as plsc

N_ATOMS = 100000
N_SEG = 4096
N_TAB = 119

NUM_TILES = 16
CHUNKS = 49
PER_TILE = CHUNKS * 128
LAST = NUM_TILES - 1
LAST_BASE = LAST * PER_TILE
LAST_N = N_ATOMS - LAST_BASE
LAST_FULL = LAST_N // 128
LAST_REM = LAST_N - LAST_FULL * 128
LAST_CHUNKS = LAST_FULL + 1
ACC = 4352
ACC_PER_TILE = ACC // NUM_TILES
OUT_PER_TILE = N_SEG // NUM_TILES
WINDOW = 16


def _body(energy_hbm, natoms_hbm, z_hbm, seg_hbm, sc_hbm, sh_hbm, table_hbm,
          out_hbm,
          table_v, z_v, seg_v, vals_v, zero_v, acc_sh,
          e_v, na_v, acc_v, res_v, s_v, sh_v, sem, sem2, sem3):
    t = lax.axis_index("s")
    base = t * PER_TILE

    @pl.when(t < LAST)
    def _():
        pltpu.async_copy(z_hbm.at[pl.ds(base, PER_TILE)], z_v, sem2)

        def stage(j, c):
            pltpu.async_copy(seg_hbm.at[pl.ds(base + j * 128, 128)],
                             seg_v.at[j], sem2)
            return c

        lax.fori_loop(0, CHUNKS, stage, 0)

    @pl.when(t == LAST)
    def _():
        pltpu.async_copy(z_hbm.at[pl.ds(LAST_BASE, LAST_N)],
                         z_v.at[pl.ds(0, LAST_N)], sem2)

        def stage(j, c):
            pltpu.async_copy(seg_hbm.at[pl.ds(LAST_BASE + j * 128, 128)],
                             seg_v.at[j], sem2)
            return c

        lax.fori_loop(0, LAST_FULL, stage, 0)
        pltpu.async_copy(seg_hbm.at[pl.ds(LAST_BASE + LAST_FULL * 128,
                                          LAST_REM)],
                         seg_v.at[LAST_FULL, pl.ds(0, LAST_REM)], sem2)

    pltpu.sync_copy(table_hbm, table_v)
    obase = t * OUT_PER_TILE
    pltpu.async_copy(energy_hbm.at[pl.ds(obase, OUT_PER_TILE)], e_v, sem3)
    pltpu.async_copy(natoms_hbm.at[pl.ds(obase, OUT_PER_TILE)], na_v, sem3)
    pltpu.sync_copy(sc_hbm, s_v)
    pltpu.sync_copy(sh_hbm, sh_v)

    for i in range(ACC_PER_TILE // 16):
        zero_v[pl.ds(i * 16, 16)] = jnp.zeros((16,), jnp.float32)
    pltpu.sync_copy(zero_v, acc_sh.at[pl.ds(t * ACC_PER_TILE, ACC_PER_TILE)])

    @pl.when(t < LAST)
    def _():
        pltpu.make_async_copy(z_hbm.at[pl.ds(0, PER_TILE)], z_v, sem2).wait()
        pltpu.make_async_copy(z_hbm.at[pl.ds(0, PER_TILE)], z_v, sem2).wait()

    @pl.when(t == LAST)
    def _():
        pltpu.make_async_copy(z_hbm.at[pl.ds(0, LAST_N)],
                              z_v.at[pl.ds(0, LAST_N)], sem2).wait()
        pltpu.make_async_copy(z_hbm.at[pl.ds(0, LAST_N)],
                              z_v.at[pl.ds(0, LAST_N)], sem2).wait()
        sink16 = jnp.full((16,), N_SEG, jnp.int32)
        zi16 = jnp.zeros((16,), jnp.int32)
        for k in range(LAST_REM // 16, 8):
            z_v[pl.ds(LAST_FULL * 128 + k * 16, 16)] = zi16
            seg_v[LAST_FULL, pl.ds(k * 16, 16)] = sink16

    plsc.subcore_barrier()

    trip = jnp.where(t == LAST, LAST_CHUNKS, CHUNKS)

    def chunk(j, c):
        for k in range(8):
            o = j * 128 + k * 16
            idx = z_v[pl.ds(o, 16)]
            vals_v[pl.ds(o, 16)] = plsc.load_gather(table_v, [idx])
        pltpu.async_copy(vals_v.at[pl.ds(j * 128, 128)],
                         acc_sh.at[seg_v.at[j]], sem, add=True)

        @pl.when(j >= WINDOW)
        def _():
            pltpu.make_async_copy(energy_hbm.at[pl.ds(0, 128)],
                                  vals_v.at[pl.ds(0, 128)], sem).wait()

        return c

    lax.fori_loop(0, trip, chunk, 0)
    for _ in range(WINDOW):
        pltpu.make_async_copy(energy_hbm.at[pl.ds(0, 128)],
                              vals_v.at[pl.ds(0, 128)], sem).wait()

    plsc.subcore_barrier()

    pltpu.make_async_copy(energy_hbm.at[pl.ds(0, OUT_PER_TILE)], e_v,
                          sem3).wait()
    pltpu.make_async_copy(natoms_hbm.at[pl.ds(0, OUT_PER_TILE)], na_v,
                          sem3).wait()
    pltpu.sync_copy(acc_sh.at[pl.ds(obase, OUT_PER_TILE)], acc_v)
    zero_i16 = jnp.zeros((16,), jnp.int32)
    s = plsc.load_gather(s_v, [zero_i16])
    sh = plsc.load_gather(sh_v, [zero_i16])
    for i in range(OUT_PER_TILE // 16):
        d = pl.ds(i * 16, 16)
        res_v[d] = e_v[d] * s + na_v[d].astype(jnp.float32) * sh + acc_v[d]
    pltpu.sync_copy(res_v, out_hbm.at[pl.ds(obase, OUT_PER_TILE)])


@jax.jit
def _run(energy, n_atoms, z, seg, scale_by, shift_by, table):
    mesh = plsc.VectorSubcoreMesh(core_axis_name="c", subcore_axis_name="s",
                                  num_cores=1)
    return pl.kernel(
        _body,
        out_type=jax.ShapeDtypeStruct((N_SEG,), jnp.float32),
        mesh=mesh,
        compiler_params=pltpu.CompilerParams(needs_layout_passes=False),
        scratch_types=[
            pltpu.VMEM((N_TAB,), jnp.float32),
            pltpu.VMEM((PER_TILE,), jnp.int32),
            pltpu.VMEM((CHUNKS, 128), jnp.int32),
            pltpu.VMEM((PER_TILE,), jnp.float32),
            pltpu.VMEM((ACC_PER_TILE,), jnp.float32),
            pltpu.VMEM_SHARED((ACC,), jnp.float32),
            pltpu.VMEM((OUT_PER_TILE,), jnp.float32),
            pltpu.VMEM((OUT_PER_TILE,), jnp.int32),
            pltpu.VMEM((OUT_PER_TILE,), jnp.float32),
            pltpu.VMEM((OUT_PER_TILE,), jnp.float32),
            pltpu.VMEM((1,), jnp.float32),
            pltpu.VMEM((1,), jnp.float32),
            pltpu.SemaphoreType.DMA,
            pltpu.SemaphoreType.DMA,
            pltpu.SemaphoreType.DMA,
        ],
    )(energy, n_atoms, z, seg, scale_by, shift_by, table)


def kernel(energy, n_atoms, Z, image_idx, scale_by, shift_by, atomic_energies):
    return _run(energy, n_atoms, Z.astype(jnp.int32),
                image_idx.astype(jnp.int32), scale_by, shift_by,
                atomic_energies)


# speedup vs baseline: 1.0721x; 1.0313x over previous
import jax
import jax.numpy as jnp
from jax import lax
from jax.experimental import pallas as pl
from jax.experimental.pallas import tpu as pltpu
from jax.experimental.pallas import tpu_sc as plsc

N_ATOMS = 100000
N_SEG = 4096
N_TAB = 119

NUM_TILES = 16
CHUNKS = 49
PER_TILE = CHUNKS * 128
LAST = NUM_TILES - 1
LAST_BASE = LAST * PER_TILE
LAST_N = N_ATOMS - LAST_BASE
LAST_FULL = LAST_N // 128
LAST_REM = LAST_N - LAST_FULL * 128
LAST_CHUNKS = LAST_FULL + 1
ACC = 4352
ACC_PER_TILE = ACC // NUM_TILES
OUT_PER_TILE = N_SEG // NUM_TILES
WINDOW = 16


def _body(energy_hbm, natoms_hbm, z_hbm, seg_hbm, sc_hbm, sh_hbm, table_hbm,
          out_hbm,
          table_v, z_v, seg_v, vals_v, zero_v, acc_sh,
          e_v, na_v, acc_v, res_v, s_v, sh_v, sem, sem2, sem3):
    t = lax.axis_index("s")
    base = t * PER_TILE

    @pl.when(t < LAST)
    def _():
        pltpu.async_copy(z_hbm.at[pl.ds(base, PER_TILE)], z_v, sem2)

        def stage(j, c):
            pltpu.async_copy(seg_hbm.at[pl.ds(base + j * 128, 128)],
                             seg_v.at[j], sem2)
            return c

        lax.fori_loop(0, CHUNKS, stage, 0)

    @pl.when(t == LAST)
    def _():
        pltpu.async_copy(z_hbm.at[pl.ds(LAST_BASE, LAST_N)],
                         z_v.at[pl.ds(0, LAST_N)], sem2)

        def stage(j, c):
            pltpu.async_copy(seg_hbm.at[pl.ds(LAST_BASE + j * 128, 128)],
                             seg_v.at[j], sem2)
            return c

        lax.fori_loop(0, LAST_FULL, stage, 0)
        pltpu.async_copy(seg_hbm.at[pl.ds(LAST_BASE + LAST_FULL * 128,
                                          LAST_REM)],
                         seg_v.at[LAST_FULL, pl.ds(0, LAST_REM)], sem2)

    pltpu.async_copy(table_hbm, table_v, sem2)
    obase = t * OUT_PER_TILE
    pltpu.async_copy(energy_hbm.at[pl.ds(obase, OUT_PER_TILE)], e_v, sem3)
    pltpu.async_copy(natoms_hbm.at[pl.ds(obase, OUT_PER_TILE)], na_v, sem3)

    for i in range(ACC_PER_TILE // 16):
        zero_v[pl.ds(i * 16, 16)] = jnp.zeros((16,), jnp.float32)
    pltpu.sync_copy(zero_v, acc_sh.at[pl.ds(t * ACC_PER_TILE, ACC_PER_TILE)])

    @pl.when(t < LAST)
    def _():
        pltpu.make_async_copy(z_hbm.at[pl.ds(0, PER_TILE)], z_v, sem2).wait()
        pltpu.make_async_copy(z_hbm.at[pl.ds(0, PER_TILE)], z_v, sem2).wait()

    @pl.when(t == LAST)
    def _():
        pltpu.make_async_copy(z_hbm.at[pl.ds(0, LAST_N)],
                              z_v.at[pl.ds(0, LAST_N)], sem2).wait()
        pltpu.make_async_copy(z_hbm.at[pl.ds(0, LAST_N)],
                              z_v.at[pl.ds(0, LAST_N)], sem2).wait()
        sink16 = jnp.full((16,), N_SEG, jnp.int32)
        zi16 = jnp.zeros((16,), jnp.int32)
        for k in range(LAST_REM // 16, 8):
            z_v[pl.ds(LAST_FULL * 128 + k * 16, 16)] = zi16
            seg_v[LAST_FULL, pl.ds(k * 16, 16)] = sink16

    pltpu.make_async_copy(table_hbm, table_v, sem2).wait()

    plsc.subcore_barrier()

    trip = jnp.where(t == LAST, LAST_CHUNKS, CHUNKS)

    def chunk(j, c):
        for k in range(8):
            o = j * 128 + k * 16
            idx = z_v[pl.ds(o, 16)]
            vals_v[pl.ds(o, 16)] = plsc.load_gather(table_v, [idx])
        pltpu.async_copy(vals_v.at[pl.ds(j * 128, 128)],
                         acc_sh.at[seg_v.at[j]], sem, add=True)

        @pl.when(j >= WINDOW)
        def _():
            pltpu.make_async_copy(energy_hbm.at[pl.ds(0, 128)],
                                  vals_v.at[pl.ds(0, 128)], sem).wait()

        return c

    lax.fori_loop(0, trip, chunk, 0)
    pltpu.sync_copy(sc_hbm, s_v)
    pltpu.sync_copy(sh_hbm, sh_v)
    for _ in range(WINDOW):
        pltpu.make_async_copy(energy_hbm.at[pl.ds(0, 128)],
                              vals_v.at[pl.ds(0, 128)], sem).wait()

    plsc.subcore_barrier()

    pltpu.make_async_copy(energy_hbm.at[pl.ds(0, OUT_PER_TILE)], e_v,
                          sem3).wait()
    pltpu.make_async_copy(natoms_hbm.at[pl.ds(0, OUT_PER_TILE)], na_v,
                          sem3).wait()
    pltpu.sync_copy(acc_sh.at[pl.ds(obase, OUT_PER_TILE)], acc_v)
    zero_i16 = jnp.zeros((16,), jnp.int32)
    s = plsc.load_gather(s_v, [zero_i16])
    sh = plsc.load_gather(sh_v, [zero_i16])
    for i in range(OUT_PER_TILE // 16):
        d = pl.ds(i * 16, 16)
        res_v[d] = e_v[d] * s + na_v[d].astype(jnp.float32) * sh + acc_v[d]
    pltpu.sync_copy(res_v, out_hbm.at[pl.ds(obase, OUT_PER_TILE)])


@jax.jit
def _run(energy, n_atoms, z, seg, scale_by, shift_by, table):
    mesh = plsc.VectorSubcoreMesh(core_axis_name="c", subcore_axis_name="s",
                                  num_cores=1)
    return pl.kernel(
        _body,
        out_type=jax.ShapeDtypeStruct((N_SEG,), jnp.float32),
        mesh=mesh,
        compiler_params=pltpu.CompilerParams(needs_layout_passes=False),
        scratch_types=[
            pltpu.VMEM((N_TAB,), jnp.float32),
            pltpu.VMEM((PER_TILE,), jnp.int32),
            pltpu.VMEM((CHUNKS, 128), jnp.int32),
            pltpu.VMEM((PER_TILE,), jnp.float32),
            pltpu.VMEM((ACC_PER_TILE,), jnp.float32),
            pltpu.VMEM_SHARED((ACC,), jnp.float32),
            pltpu.VMEM((OUT_PER_TILE,), jnp.float32),
            pltpu.VMEM((OUT_PER_TILE,), jnp.int32),
            pltpu.VMEM((OUT_PER_TILE,), jnp.float32),
            pltpu.VMEM((OUT_PER_TILE,), jnp.float32),
            pltpu.VMEM((1,), jnp.float32),
            pltpu.VMEM((1,), jnp.float32),
            pltpu.SemaphoreType.DMA,
            pltpu.SemaphoreType.DMA,
            pltpu.SemaphoreType.DMA,
        ],
    )(energy, n_atoms, z, seg, scale_by, shift_by, table)


def kernel(energy, n_atoms, Z, image_idx, scale_by, shift_by, atomic_energies):
    return _run(energy, n_atoms, Z.astype(jnp.int32),
                image_idx.astype(jnp.int32), scale_by, shift_by,
                atomic_energies)
